# Initial kernel scaffold; baseline (speedup 1.0000x reference)
import jax
import jax.numpy as jnp
from jax.experimental import pallas as pl


def kernel(node_feature, one_adj_list, two_adj_list, W11, b11, W12, b12, W21, b21, W22, b22, W31, b31, W32, b32, Wa1, ba1, Wa2, ba2, Wa3, ba3):
    raise NotImplementedError("write your pallas kernel here")


# speedup vs baseline: 7.4294x; 7.4294x over previous
import functools

import jax
import jax.numpy as jnp
from jax import lax
from jax.experimental import pallas as pl
from jax.experimental.pallas import tpu as pltpu
from jax.experimental.pallas import tpu_sc as plsc

_NUM_SC = 2
_NUM_TILES = 16
_NW = _NUM_SC * _NUM_TILES
_B = 40


def _elu(x):
    return jnp.where(x > 0.0, x, jnp.exp(x) - 1.0)


@functools.lru_cache(maxsize=None)
def _make_segsum(n, e, d, num_slabs, gather):
    ew = e // _NW
    nb = ew // _B
    stripe = 640
    nacc = _NUM_TILES * stripe
    zr = 40
    nzc = stripe // zr
    ncols = d // 16

    mesh = plsc.VectorSubcoreMesh(
        core_axis_name="c", subcore_axis_name="s",
        num_cores=_NUM_SC, num_subcores=_NUM_TILES)

    def body(*refs):
        it = iter(refs)
        if gather:
            src_hbm = next(it)
        dst_hbm = next(it)
        h_hbm = [next(it) for _ in range(num_slabs)] if gather else []
        out_hbm = [next(it) for _ in range(num_slabs)]
        acc = next(it)
        if gather:
            src_v = next(it)
        dst_v = next(it)
        rows_v = next(it)
        zbuf = next(it)
        dbuf = next(it)
        sem = next(it)

        c = lax.axis_index("c")
        s = lax.axis_index("s")
        w = c * _NUM_TILES + s
        base_row = s * stripe

        zero16 = jnp.zeros((16,), jnp.float32)
        for i in range(zr):
            for j in range(ncols):
                zbuf[i, pl.ds(j * 16, 16)] = zero16

        if not gather:
            one16 = jnp.ones((16,), jnp.float32)
            for i in range(_B):
                for j in range(ncols):
                    rows_v[i, pl.ds(j * 16, 16)] = one16

        if gather:
            pltpu.sync_copy(src_hbm.at[w], src_v)
        pltpu.sync_copy(dst_hbm.at[w], dst_v)

        for slab in range(num_slabs):
            for k in range(nzc):
                pltpu.sync_copy(zbuf, acc.at[pl.ds(base_row + k * zr, zr)])
            plsc.subcore_barrier()

            if gather:
                h_slab = h_hbm[slab]

                def step(j, _):
                    pltpu.async_copy(h_slab.at[src_v.at[j]], rows_v, sem).wait()
                    pltpu.sync_copy(rows_v, acc.at[dst_v.at[j]], add=True)
                    return 0
            else:
                def step(j, _):
                    pltpu.sync_copy(rows_v, acc.at[dst_v.at[j]], add=True)
                    return 0
            lax.fori_loop(0, nb, step, 0)
            plsc.subcore_barrier()

            o_slab = out_hbm[slab]
            for k in range(nzc):
                pltpu.sync_copy(acc.at[pl.ds(base_row + k * zr, zr)], dbuf)
                pltpu.sync_copy(dbuf, o_slab.at[c, pl.ds(base_row + k * zr, zr)])

    out_type = [jax.ShapeDtypeStruct((_NUM_SC, nacc, d), jnp.float32)
                for _ in range(num_slabs)]
    scratch = [pltpu.VMEM_SHARED((nacc, d), jnp.float32)]
    if gather:
        scratch.append(pltpu.VMEM((nb, _B), jnp.int32))
    scratch += [
        pltpu.VMEM((nb, _B), jnp.int32),
        pltpu.VMEM((_B, d), jnp.float32),
        pltpu.VMEM((zr, d), jnp.float32),
        pltpu.VMEM((zr, d), jnp.float32),
        pltpu.SemaphoreType.DMA,
    ]
    return pl.kernel(body, out_type=out_type, mesh=mesh, scratch_types=scratch)


def _segsum(src, dst, h_slabs, n, e, d):
    fn = _make_segsum(n, e, d, len(h_slabs), True)
    return [o[:, :n, :] for o in fn(src, dst, *h_slabs)]


def _degrees(dst, n, e):
    fn = _make_segsum(n, e, 128, 1, False)
    return fn(dst)[0][:, :n, :]


_RB = 1000


def _full(shape):
    return pl.BlockSpec(shape, lambda i: (0,) * len(shape))


def _rows(*lead):
    def mk(shape):
        nl = len(lead)
        return pl.BlockSpec(tuple(lead) + shape,
                            lambda i: (0,) * nl + (i,) + (0,) * (len(shape) - 1))
    return mk


def _tc_call(body, nblk, out_shapes, in_specs, out_specs):
    return pl.pallas_call(
        body,
        grid=(nblk,),
        in_specs=in_specs,
        out_specs=out_specs,
        out_shape=out_shapes,
        compiler_params=pltpu.CompilerParams(
            dimension_semantics=("arbitrary",)),
    )


def _prep(deg1, deg2, x, n):
    def body(q1, q2, xr, dv1, dv2, xs1, xs2):
        d1 = lax.rsqrt(q1[0, :, :1] + q1[1, :, :1] + 1.0)
        d2 = lax.rsqrt(q2[0, :, :1] + q2[1, :, :1] + 1.0)
        dv1[...] = d1
        dv2[...] = d2
        xs1[...] = d1 * xr[...]
        xs2[...] = d2 * xr[...]

    nblk = n // _RB
    f = x.shape[1]
    return _tc_call(
        body, nblk,
        [jax.ShapeDtypeStruct((n, 1), jnp.float32),
         jax.ShapeDtypeStruct((n, 1), jnp.float32),
         jax.ShapeDtypeStruct((n, f), jnp.float32),
         jax.ShapeDtypeStruct((n, f), jnp.float32)],
        [_rows(_NUM_SC)((_RB, 128)), _rows(_NUM_SC)((_RB, 128)),
         _rows()((_RB, f))],
        [_rows()((_RB, 1)), _rows()((_RB, 1)),
         _rows()((_RB, f)), _rows()((_RB, f))],
    )(deg1, deg2, x)


def _layer1(p1, xs1, dv1, p2, xs2, dv2, W11, b11, W12, b12, Wa, ba, n):
    f = xs1.shape[1]
    k1 = W11.shape[1]
    m = Wa.shape[1]

    def body(p1r, xs1r, d1r, p2r, xs2r, d2r, w11, bb11, w12, bb12, wa, bba, out):
        a1 = d1r[...] * (p1r[0] + p1r[1] + xs1r[...])
        t1 = _elu(jnp.dot(a1, w11[...], preferred_element_type=jnp.float32)
                  + bb11[...])
        a2 = d2r[...] * (p2r[0] + p2r[1] + xs2r[...])
        t2 = _elu(jnp.dot(a2, w12[...], preferred_element_type=jnp.float32)
                  + bb12[...])
        out[...] = (jnp.dot(t1, wa[:k1, :], preferred_element_type=jnp.float32)
                    + jnp.dot(t2, wa[k1:, :], preferred_element_type=jnp.float32)
                    + bba[...])

    nblk = n // _RB
    return _tc_call(
        body, nblk,
        jax.ShapeDtypeStruct((n, m), jnp.float32),
        [_rows(_NUM_SC)((_RB, f)), _rows()((_RB, f)), _rows()((_RB, 1)),
         _rows(_NUM_SC)((_RB, f)), _rows()((_RB, f)), _rows()((_RB, 1)),
         _full((f, k1)), _full((1, k1)), _full((f, k1)), _full((1, k1)),
         _full((2 * k1, m)), _full((1, m))],
        _rows()((_RB, m)),
    )(p1, xs1, dv1, p2, xs2, dv2, W11, b11, W12, b12, Wa, ba)


def _branch_mm(m1, Wb1, Wb2, dv1, dv2, n):
    k = Wb1.shape[0]
    mout = Wb1.shape[1]
    ns = mout // 128
    nblk = n // _RB

    def body(mr, w1, w2, d1r, d2r, o1, o2):
        o1[0] = d1r[...] * jnp.dot(mr[...], w1[...],
                                   preferred_element_type=jnp.float32)
        o2[0] = d2r[...] * jnp.dot(mr[...], w2[...],
                                   preferred_element_type=jnp.float32)

    def colblk(shape):
        return pl.BlockSpec(shape, lambda i, j: (0, j))

    return pl.pallas_call(
        body,
        grid=(nblk, ns),
        in_specs=[pl.BlockSpec((_RB, k), lambda i, j: (i, 0)),
                  colblk((k, 128)), colblk((k, 128)),
                  pl.BlockSpec((_RB, 1), lambda i, j: (i, 0)),
                  pl.BlockSpec((_RB, 1), lambda i, j: (i, 0))],
        out_specs=[pl.BlockSpec((1, _RB, 128), lambda i, j: (j, i, 0)),
                   pl.BlockSpec((1, _RB, 128), lambda i, j: (j, i, 0))],
        out_shape=[jax.ShapeDtypeStruct((ns, n, 128), jnp.float32),
                   jax.ShapeDtypeStruct((ns, n, 128), jnp.float32)],
        compiler_params=pltpu.CompilerParams(
            dimension_semantics=("arbitrary", "arbitrary")),
    )(m1, Wb1, Wb2, dv1, dv2)


def _combine_mm(q1, hs1, dv1, b1, q2, hs2, dv2, b2, Wa, ba, n):
    ns = hs1.shape[0]
    m = Wa.shape[1]
    nblk = n // _RB

    def body(*refs):
        it = iter(refs)
        q1r = [next(it) for _ in range(ns)]
        hs1r, d1r, b1r = next(it), next(it), next(it)
        q2r = [next(it) for _ in range(ns)]
        hs2r, d2r, b2r = next(it), next(it), next(it)
        wa, bar, out = next(it), next(it), next(it)

        acc = jnp.zeros((_RB, m), jnp.float32) + bar[...]
        for r, (qr, hsr, dr, br) in enumerate(
                ((q1r, hs1r, d1r, b1r), (q2r, hs2r, d2r, b2r))):
            for sl in range(ns):
                t = _elu(dr[...] * (qr[sl][0] + qr[sl][1] + hsr[sl])
                         + br[:, sl * 128:(sl + 1) * 128])
                acc += jnp.dot(t, wa[(r * ns + sl) * 128:(r * ns + sl + 1) * 128, :],
                               preferred_element_type=jnp.float32)
        out[...] = acc

    specs = ([_rows(_NUM_SC)((_RB, 128))] * ns
             + [_rows(ns)((_RB, 128)), _rows()((_RB, 1)), _full((1, ns * 128))])
    in_specs = (specs
                + [_rows(_NUM_SC)((_RB, 128))] * ns
                + [_rows(ns)((_RB, 128)), _rows()((_RB, 1)), _full((1, ns * 128))]
                + [_full((2 * ns * 128, m)), _full((1, m))])
    return _tc_call(
        body, nblk,
        jax.ShapeDtypeStruct((n, m), jnp.float32),
        in_specs,
        _rows()((_RB, m)),
    )(*q1, hs1, dv1, b1, *q2, hs2, dv2, b2, Wa, ba)


def kernel(node_feature, one_adj_list, two_adj_list,
                 W11, b11, W12, b12, W21, b21, W22, b22, W31, b31, W32, b32,
                 Wa1, ba1, Wa2, ba2, Wa3, ba3):
    n, f = node_feature.shape
    e = one_adj_list.shape[1]
    ew = e // _NW

    def _edges(adj):
        a = adj.astype(jnp.int32)
        return (a[0].reshape(_NW, ew // _B, _B), a[1].reshape(_NW, ew // _B, _B))

    src1, dst1 = _edges(one_adj_list)
    src2, dst2 = _edges(two_adj_list)

    def row2d(b):
        return b.reshape(1, -1)

    deg1 = _degrees(dst1, n, e)
    deg2 = _degrees(dst2, n, e)
    dv1, dv2, xs1, xs2 = _prep(deg1, deg2, node_feature, n)

    p1 = _segsum(src1, dst1, [xs1], n, e, f)[0]
    p2 = _segsum(src2, dst2, [xs2], n, e, f)[0]
    m1 = _layer1(p1, xs1, dv1, p2, xs2, dv2,
                 W11, row2d(b11), W12, row2d(b12), Wa1, row2d(ba1), n)

    hs1, hs2 = _branch_mm(m1, W21, W22, dv1, dv2, n)
    ns2 = hs1.shape[0]
    q1 = _segsum(src1, dst1, [hs1[i] for i in range(ns2)], n, e, 128)
    q2 = _segsum(src2, dst2, [hs2[i] for i in range(ns2)], n, e, 128)
    m2 = _combine_mm(q1, hs1, dv1, row2d(b21), q2, hs2, dv2, row2d(b22),
                     Wa2, row2d(ba2), n)

    h31, h32 = _branch_mm(m2, W31, W32, dv1, dv2, n)
    q31 = _segsum(src1, dst1, [h31[0]], n, e, 128)
    q32 = _segsum(src2, dst2, [h32[0]], n, e, 128)
    out = _combine_mm(q31, h31, dv1, row2d(b31), q32, h32, dv2, row2d(b32),
                      Wa3, row2d(ba3), n)
    return out


# speedup vs baseline: 10.6962x; 1.4397x over previous
import functools

import jax
import jax.numpy as jnp
from jax import lax
from jax.experimental import pallas as pl
from jax.experimental.pallas import tpu as pltpu
from jax.experimental.pallas import tpu_sc as plsc

_NUM_SC = 2
_NUM_TILES = 16
_NW = _NUM_SC * _NUM_TILES
_B = 125


def _elu(x):
    return jnp.where(x > 0.0, x, jnp.exp(x) - 1.0)


@functools.lru_cache(maxsize=None)
def _make_segsum(n, e, d, num_slabs, gather):
    ew = e // _NW
    nb = ew // _B
    stripe = 640
    nacc = _NUM_TILES * stripe
    zr = 40
    nzc = stripe // zr
    ncols = d // 16

    mesh = plsc.VectorSubcoreMesh(
        core_axis_name="c", subcore_axis_name="s",
        num_cores=_NUM_SC, num_subcores=_NUM_TILES)

    def body(*refs):
        it = iter(refs)
        if gather:
            src_hbm = next(it)
        dst_hbm = next(it)
        h_hbm = [next(it) for _ in range(num_slabs)] if gather else []
        out_hbm = [next(it) for _ in range(num_slabs)]
        acc = next(it)
        if gather:
            src_v = next(it)
        dst_v = next(it)
        rows_v = next(it)
        zbuf = next(it)
        dbuf = next(it)
        sem = next(it)

        c = lax.axis_index("c")
        s = lax.axis_index("s")
        w = c * _NUM_TILES + s
        base_row = s * stripe

        zero16 = jnp.zeros((16,), jnp.float32)
        for i in range(zr):
            for j in range(ncols):
                zbuf[i, pl.ds(j * 16, 16)] = zero16

        if not gather:
            one16 = jnp.ones((16,), jnp.float32)
            for i in range(_B):
                for j in range(ncols):
                    rows_v[i, pl.ds(j * 16, 16)] = one16

        if gather:
            pltpu.sync_copy(src_hbm.at[w], src_v)
        pltpu.sync_copy(dst_hbm.at[w], dst_v)

        for slab in range(num_slabs):
            for k in range(nzc):
                pltpu.sync_copy(zbuf, acc.at[pl.ds(base_row + k * zr, zr)])
            plsc.subcore_barrier()

            if gather:
                h_slab = h_hbm[slab]

                def step(j, _):
                    pltpu.async_copy(h_slab.at[src_v.at[j]], rows_v, sem).wait()
                    pltpu.sync_copy(rows_v, acc.at[dst_v.at[j]], add=True)
                    return 0
            else:
                def step(j, _):
                    pltpu.sync_copy(rows_v, acc.at[dst_v.at[j]], add=True)
                    return 0
            lax.fori_loop(0, nb, step, 0)
            plsc.subcore_barrier()

            o_slab = out_hbm[slab]
            for k in range(nzc):
                pltpu.sync_copy(acc.at[pl.ds(base_row + k * zr, zr)], dbuf)
                pltpu.sync_copy(dbuf, o_slab.at[c, pl.ds(base_row + k * zr, zr)])

    out_type = [jax.ShapeDtypeStruct((_NUM_SC, nacc, d), jnp.float32)
                for _ in range(num_slabs)]
    scratch = [pltpu.VMEM_SHARED((nacc, d), jnp.float32)]
    if gather:
        scratch.append(pltpu.VMEM((nb, _B), jnp.int32))
    scratch += [
        pltpu.VMEM((nb, _B), jnp.int32),
        pltpu.VMEM((_B, d), jnp.float32),
        pltpu.VMEM((zr, d), jnp.float32),
        pltpu.VMEM((zr, d), jnp.float32),
        pltpu.SemaphoreType.DMA,
    ]
    return pl.kernel(body, out_type=out_type, mesh=mesh, scratch_types=scratch)


def _segsum(src, dst, h_slabs, n, e, d):
    fn = _make_segsum(n, e, d, len(h_slabs), True)
    return [o[:, :n, :] for o in fn(src, dst, *h_slabs)]


def _degrees(dst, n, e):
    fn = _make_segsum(n, e, 128, 1, False)
    return fn(dst)[0][:, :n, :]


_RB = 1000


def _full(shape):
    return pl.BlockSpec(shape, lambda i: (0,) * len(shape))


def _rows(*lead):
    def mk(shape):
        nl = len(lead)
        return pl.BlockSpec(tuple(lead) + shape,
                            lambda i: (0,) * nl + (i,) + (0,) * (len(shape) - 1))
    return mk


def _tc_call(body, nblk, out_shapes, in_specs, out_specs):
    return pl.pallas_call(
        body,
        grid=(nblk,),
        in_specs=in_specs,
        out_specs=out_specs,
        out_shape=out_shapes,
        compiler_params=pltpu.CompilerParams(
            dimension_semantics=("arbitrary",)),
    )


def _prep(deg1, deg2, x, n):
    def body(q1, q2, xr, dv1, dv2, xs1, xs2):
        d1 = lax.rsqrt(q1[0, :, :1] + q1[1, :, :1] + 1.0)
        d2 = lax.rsqrt(q2[0, :, :1] + q2[1, :, :1] + 1.0)
        dv1[...] = d1
        dv2[...] = d2
        xs1[...] = d1 * xr[...]
        xs2[...] = d2 * xr[...]

    nblk = n // _RB
    f = x.shape[1]
    return _tc_call(
        body, nblk,
        [jax.ShapeDtypeStruct((n, 1), jnp.float32),
         jax.ShapeDtypeStruct((n, 1), jnp.float32),
         jax.ShapeDtypeStruct((n, f), jnp.float32),
         jax.ShapeDtypeStruct((n, f), jnp.float32)],
        [_rows(_NUM_SC)((_RB, 128)), _rows(_NUM_SC)((_RB, 128)),
         _rows()((_RB, f))],
        [_rows()((_RB, 1)), _rows()((_RB, 1)),
         _rows()((_RB, f)), _rows()((_RB, f))],
    )(deg1, deg2, x)


def _layer1(p1, xs1, dv1, p2, xs2, dv2, W11, b11, W12, b12, Wa, ba, n):
    f = xs1.shape[1]
    k1 = W11.shape[1]
    m = Wa.shape[1]

    def body(p1r, xs1r, d1r, p2r, xs2r, d2r, w11, bb11, w12, bb12, wa, bba, out):
        a1 = d1r[...] * (p1r[0] + p1r[1] + xs1r[...])
        t1 = _elu(jnp.dot(a1, w11[...], preferred_element_type=jnp.float32)
                  + bb11[...])
        a2 = d2r[...] * (p2r[0] + p2r[1] + xs2r[...])
        t2 = _elu(jnp.dot(a2, w12[...], preferred_element_type=jnp.float32)
                  + bb12[...])
        out[...] = (jnp.dot(t1, wa[:k1, :], preferred_element_type=jnp.float32)
                    + jnp.dot(t2, wa[k1:, :], preferred_element_type=jnp.float32)
                    + bba[...])

    nblk = n // _RB
    return _tc_call(
        body, nblk,
        jax.ShapeDtypeStruct((n, m), jnp.float32),
        [_rows(_NUM_SC)((_RB, f)), _rows()((_RB, f)), _rows()((_RB, 1)),
         _rows(_NUM_SC)((_RB, f)), _rows()((_RB, f)), _rows()((_RB, 1)),
         _full((f, k1)), _full((1, k1)), _full((f, k1)), _full((1, k1)),
         _full((2 * k1, m)), _full((1, m))],
        _rows()((_RB, m)),
    )(p1, xs1, dv1, p2, xs2, dv2, W11, b11, W12, b12, Wa, ba)


def _branch_mm(m1, Wb1, Wb2, dv1, dv2, n):
    k = Wb1.shape[0]
    mout = Wb1.shape[1]
    ns = mout // 128
    nblk = n // _RB

    def body(mr, w1, w2, d1r, d2r, o1, o2):
        o1[0] = d1r[...] * jnp.dot(mr[...], w1[...],
                                   preferred_element_type=jnp.float32)
        o2[0] = d2r[...] * jnp.dot(mr[...], w2[...],
                                   preferred_element_type=jnp.float32)

    def colblk(shape):
        return pl.BlockSpec(shape, lambda i, j: (0, j))

    return pl.pallas_call(
        body,
        grid=(nblk, ns),
        in_specs=[pl.BlockSpec((_RB, k), lambda i, j: (i, 0)),
                  colblk((k, 128)), colblk((k, 128)),
                  pl.BlockSpec((_RB, 1), lambda i, j: (i, 0)),
                  pl.BlockSpec((_RB, 1), lambda i, j: (i, 0))],
        out_specs=[pl.BlockSpec((1, _RB, 128), lambda i, j: (j, i, 0)),
                   pl.BlockSpec((1, _RB, 128), lambda i, j: (j, i, 0))],
        out_shape=[jax.ShapeDtypeStruct((ns, n, 128), jnp.float32),
                   jax.ShapeDtypeStruct((ns, n, 128), jnp.float32)],
        compiler_params=pltpu.CompilerParams(
            dimension_semantics=("arbitrary", "arbitrary")),
    )(m1, Wb1, Wb2, dv1, dv2)


def _combine_mm(q1, hs1, dv1, b1, q2, hs2, dv2, b2, Wa, ba, n):
    ns = hs1.shape[0]
    m = Wa.shape[1]
    nblk = n // _RB

    def body(*refs):
        it = iter(refs)
        q1r = [next(it) for _ in range(ns)]
        hs1r, d1r, b1r = next(it), next(it), next(it)
        q2r = [next(it) for _ in range(ns)]
        hs2r, d2r, b2r = next(it), next(it), next(it)
        wa, bar, out = next(it), next(it), next(it)

        acc = jnp.zeros((_RB, m), jnp.float32) + bar[...]
        for r, (qr, hsr, dr, br) in enumerate(
                ((q1r, hs1r, d1r, b1r), (q2r, hs2r, d2r, b2r))):
            for sl in range(ns):
                t = _elu(dr[...] * (qr[sl][0] + qr[sl][1] + hsr[sl])
                         + br[:, sl * 128:(sl + 1) * 128])
                acc += jnp.dot(t, wa[(r * ns + sl) * 128:(r * ns + sl + 1) * 128, :],
                               preferred_element_type=jnp.float32)
        out[...] = acc

    specs = ([_rows(_NUM_SC)((_RB, 128))] * ns
             + [_rows(ns)((_RB, 128)), _rows()((_RB, 1)), _full((1, ns * 128))])
    in_specs = (specs
                + [_rows(_NUM_SC)((_RB, 128))] * ns
                + [_rows(ns)((_RB, 128)), _rows()((_RB, 1)), _full((1, ns * 128))]
                + [_full((2 * ns * 128, m)), _full((1, m))])
    return _tc_call(
        body, nblk,
        jax.ShapeDtypeStruct((n, m), jnp.float32),
        in_specs,
        _rows()((_RB, m)),
    )(*q1, hs1, dv1, b1, *q2, hs2, dv2, b2, Wa, ba)


def kernel(node_feature, one_adj_list, two_adj_list,
                 W11, b11, W12, b12, W21, b21, W22, b22, W31, b31, W32, b32,
                 Wa1, ba1, Wa2, ba2, Wa3, ba3):
    n, f = node_feature.shape
    e = one_adj_list.shape[1]
    ew = e // _NW

    def _edges(adj):
        a = adj.astype(jnp.int32)
        return (a[0].reshape(_NW, ew // _B, _B), a[1].reshape(_NW, ew // _B, _B))

    src1, dst1 = _edges(one_adj_list)
    src2, dst2 = _edges(two_adj_list)

    def row2d(b):
        return b.reshape(1, -1)

    deg1 = _degrees(dst1, n, e)
    deg2 = _degrees(dst2, n, e)
    dv1, dv2, xs1, xs2 = _prep(deg1, deg2, node_feature, n)

    p1 = _segsum(src1, dst1, [xs1], n, e, f)[0]
    p2 = _segsum(src2, dst2, [xs2], n, e, f)[0]
    m1 = _layer1(p1, xs1, dv1, p2, xs2, dv2,
                 W11, row2d(b11), W12, row2d(b12), Wa1, row2d(ba1), n)

    hs1, hs2 = _branch_mm(m1, W21, W22, dv1, dv2, n)
    ns2 = hs1.shape[0]
    q1 = _segsum(src1, dst1, [hs1[i] for i in range(ns2)], n, e, 128)
    q2 = _segsum(src2, dst2, [hs2[i] for i in range(ns2)], n, e, 128)
    m2 = _combine_mm(q1, hs1, dv1, row2d(b21), q2, hs2, dv2, row2d(b22),
                     Wa2, row2d(ba2), n)

    h31, h32 = _branch_mm(m2, W31, W32, dv1, dv2, n)
    q31 = _segsum(src1, dst1, [h31[0]], n, e, 128)
    q32 = _segsum(src2, dst2, [h32[0]], n, e, 128)
    out = _combine_mm(q31, h31, dv1, row2d(b31), q32, h32, dv2, row2d(b32),
                      Wa3, row2d(ba3), n)
    return out


# speedup vs baseline: 12.1422x; 1.1352x over previous
import functools

import jax
import jax.numpy as jnp
from jax import lax
from jax.experimental import pallas as pl
from jax.experimental.pallas import tpu as pltpu
from jax.experimental.pallas import tpu_sc as plsc

_NUM_SC = 2
_NUM_TILES = 16
_NW = _NUM_SC * _NUM_TILES
_B = 100


def _elu(x):
    return jnp.where(x > 0.0, x, jnp.exp(x) - 1.0)


_NBUF = 2


@functools.lru_cache(maxsize=None)
def _make_segsum(n, e, d, num_slabs, gather):
    ew = e // _NW
    nb = ew // _B
    stripe = 640
    nacc = _NUM_TILES * stripe
    zr = 32
    nzc = stripe // zr
    ncols = d // 16

    mesh = plsc.VectorSubcoreMesh(
        core_axis_name="c", subcore_axis_name="s",
        num_cores=_NUM_SC, num_subcores=_NUM_TILES)

    def body(*refs):
        it = iter(refs)
        if gather:
            src_hbm = next(it)
        dst_hbm = next(it)
        h_hbm = [next(it) for _ in range(num_slabs)] if gather else []
        out_hbm = [next(it) for _ in range(num_slabs)]
        acc = next(it)
        if gather:
            src_v = next(it)
        dst_v = next(it)
        bufs = [next(it) for _ in range(_NBUF)]
        dbufs = [next(it) for _ in range(2)]
        zbuf = dbufs[0]
        gsems = [next(it) for _ in range(_NBUF)]
        ssems = [next(it) for _ in range(_NBUF)]
        zsem = next(it)
        dsems = [next(it) for _ in range(2)]
        osems = [next(it) for _ in range(2)]

        c = lax.axis_index("c")
        s = lax.axis_index("s")
        w = c * _NUM_TILES + s
        base_row = s * stripe

        zero16 = jnp.zeros((16,), jnp.float32)

        if not gather:
            one16 = jnp.ones((16,), jnp.float32)
            for i in range(_B):
                for j in range(ncols):
                    bufs[0][i, pl.ds(j * 16, 16)] = one16

        if gather:
            pltpu.sync_copy(src_hbm.at[w], src_v)
        pltpu.sync_copy(dst_hbm.at[w], dst_v)

        for slab in range(num_slabs):
            for i in range(zr):
                for j in range(ncols):
                    zbuf[i, pl.ds(j * 16, 16)] = zero16
            zds = [pltpu.async_copy(zbuf, acc.at[pl.ds(base_row + k * zr, zr)],
                                    zsem)
                   for k in range(nzc)]
            for zd in zds:
                zd.wait()
            plsc.subcore_barrier()

            if gather:
                h_slab = h_hbm[slab]

                def step(jq, _):
                    j0 = jq * _NBUF
                    gds = [pltpu.async_copy(h_slab.at[src_v.at[j0 + b]],
                                            bufs[b], gsems[b])
                           for b in range(_NBUF)]
                    sds = []
                    for b in range(_NBUF):
                        gds[b].wait()
                        sds.append(pltpu.async_copy(
                            bufs[b], acc.at[dst_v.at[j0 + b]], ssems[b],
                            add=True))
                    for sd in sds:
                        sd.wait()
                    return 0
            else:
                def step(jq, _):
                    j0 = jq * _NBUF
                    sds = [pltpu.async_copy(bufs[0], acc.at[dst_v.at[j0 + b]],
                                            ssems[b], add=True)
                           for b in range(_NBUF)]
                    for sd in sds:
                        sd.wait()
                    return 0
            lax.fori_loop(0, nb // _NBUF, step, 0)
            plsc.subcore_barrier()

            o_slab = out_hbm[slab]
            sdesc = [None, None]
            odesc = [None, None]
            for k in range(nzc):
                p = k % 2
                if odesc[p] is not None:
                    odesc[p].wait()
                sdesc[p] = pltpu.async_copy(
                    acc.at[pl.ds(base_row + k * zr, zr)], dbufs[p], dsems[p])
                sdesc[p].wait()
                odesc[p] = pltpu.async_copy(
                    dbufs[p], o_slab.at[c, pl.ds(base_row + k * zr, zr)],
                    osems[p])
            for od in odesc:
                if od is not None:
                    od.wait()

    out_type = [jax.ShapeDtypeStruct((_NUM_SC, nacc, d), jnp.float32)
                for _ in range(num_slabs)]
    scratch = [pltpu.VMEM_SHARED((nacc, d), jnp.float32)]
    if gather:
        scratch.append(pltpu.VMEM((nb, _B), jnp.int32))
    scratch += [pltpu.VMEM((nb, _B), jnp.int32)]
    scratch += [pltpu.VMEM((_B, d), jnp.float32) for _ in range(_NBUF)]
    scratch += [pltpu.VMEM((zr, d), jnp.float32) for _ in range(2)]
    scratch += [pltpu.SemaphoreType.DMA] * (2 * _NBUF + 5)
    return pl.kernel(body, out_type=out_type, mesh=mesh, scratch_types=scratch)


def _segsum(src, dst, h_slabs, n, e, d):
    fn = _make_segsum(n, e, d, len(h_slabs), True)
    return [o[:, :n, :] for o in fn(src, dst, *h_slabs)]


def _degrees(dst, n, e):
    fn = _make_segsum(n, e, 128, 1, False)
    return fn(dst)[0][:, :n, :]


_RB = 1000


def _full(shape):
    return pl.BlockSpec(shape, lambda i: (0,) * len(shape))


def _rows(*lead):
    def mk(shape):
        nl = len(lead)
        return pl.BlockSpec(tuple(lead) + shape,
                            lambda i: (0,) * nl + (i,) + (0,) * (len(shape) - 1))
    return mk


def _tc_call(body, nblk, out_shapes, in_specs, out_specs):
    return pl.pallas_call(
        body,
        grid=(nblk,),
        in_specs=in_specs,
        out_specs=out_specs,
        out_shape=out_shapes,
        compiler_params=pltpu.CompilerParams(
            dimension_semantics=("arbitrary",)),
    )


def _prep(deg1, deg2, x, n):
    def body(q1, q2, xr, dv1, dv2, xs1, xs2):
        d1 = lax.rsqrt(q1[0, :, :1] + q1[1, :, :1] + 1.0)
        d2 = lax.rsqrt(q2[0, :, :1] + q2[1, :, :1] + 1.0)
        dv1[...] = d1
        dv2[...] = d2
        xs1[...] = d1 * xr[...]
        xs2[...] = d2 * xr[...]

    nblk = n // _RB
    f = x.shape[1]
    return _tc_call(
        body, nblk,
        [jax.ShapeDtypeStruct((n, 1), jnp.float32),
         jax.ShapeDtypeStruct((n, 1), jnp.float32),
         jax.ShapeDtypeStruct((n, f), jnp.float32),
         jax.ShapeDtypeStruct((n, f), jnp.float32)],
        [_rows(_NUM_SC)((_RB, 128)), _rows(_NUM_SC)((_RB, 128)),
         _rows()((_RB, f))],
        [_rows()((_RB, 1)), _rows()((_RB, 1)),
         _rows()((_RB, f)), _rows()((_RB, f))],
    )(deg1, deg2, x)


def _layer1(p1, xs1, dv1, p2, xs2, dv2, W11, b11, W12, b12, Wa, ba, n):
    f = xs1.shape[1]
    k1 = W11.shape[1]
    m = Wa.shape[1]

    def body(p1r, xs1r, d1r, p2r, xs2r, d2r, w11, bb11, w12, bb12, wa, bba, out):
        a1 = d1r[...] * (p1r[0] + p1r[1] + xs1r[...])
        t1 = _elu(jnp.dot(a1, w11[...], preferred_element_type=jnp.float32)
                  + bb11[...])
        a2 = d2r[...] * (p2r[0] + p2r[1] + xs2r[...])
        t2 = _elu(jnp.dot(a2, w12[...], preferred_element_type=jnp.float32)
                  + bb12[...])
        out[...] = (jnp.dot(t1, wa[:k1, :], preferred_element_type=jnp.float32)
                    + jnp.dot(t2, wa[k1:, :], preferred_element_type=jnp.float32)
                    + bba[...])

    nblk = n // _RB
    return _tc_call(
        body, nblk,
        jax.ShapeDtypeStruct((n, m), jnp.float32),
        [_rows(_NUM_SC)((_RB, f)), _rows()((_RB, f)), _rows()((_RB, 1)),
         _rows(_NUM_SC)((_RB, f)), _rows()((_RB, f)), _rows()((_RB, 1)),
         _full((f, k1)), _full((1, k1)), _full((f, k1)), _full((1, k1)),
         _full((2 * k1, m)), _full((1, m))],
        _rows()((_RB, m)),
    )(p1, xs1, dv1, p2, xs2, dv2, W11, b11, W12, b12, Wa, ba)


def _branch_mm(m1, Wb1, Wb2, dv1, dv2, n):
    k = Wb1.shape[0]
    mout = Wb1.shape[1]
    ns = mout // 128
    nblk = n // _RB

    def body(mr, w1, w2, d1r, d2r, o1, o2):
        o1[0] = d1r[...] * jnp.dot(mr[...], w1[...],
                                   preferred_element_type=jnp.float32)
        o2[0] = d2r[...] * jnp.dot(mr[...], w2[...],
                                   preferred_element_type=jnp.float32)

    def colblk(shape):
        return pl.BlockSpec(shape, lambda i, j: (0, j))

    return pl.pallas_call(
        body,
        grid=(nblk, ns),
        in_specs=[pl.BlockSpec((_RB, k), lambda i, j: (i, 0)),
                  colblk((k, 128)), colblk((k, 128)),
                  pl.BlockSpec((_RB, 1), lambda i, j: (i, 0)),
                  pl.BlockSpec((_RB, 1), lambda i, j: (i, 0))],
        out_specs=[pl.BlockSpec((1, _RB, 128), lambda i, j: (j, i, 0)),
                   pl.BlockSpec((1, _RB, 128), lambda i, j: (j, i, 0))],
        out_shape=[jax.ShapeDtypeStruct((ns, n, 128), jnp.float32),
                   jax.ShapeDtypeStruct((ns, n, 128), jnp.float32)],
        compiler_params=pltpu.CompilerParams(
            dimension_semantics=("arbitrary", "arbitrary")),
    )(m1, Wb1, Wb2, dv1, dv2)


def _combine_mm(q1, hs1, dv1, b1, q2, hs2, dv2, b2, Wa, ba, n):
    ns = hs1.shape[0]
    m = Wa.shape[1]
    nblk = n // _RB

    def body(*refs):
        it = iter(refs)
        q1r = [next(it) for _ in range(ns)]
        hs1r, d1r, b1r = next(it), next(it), next(it)
        q2r = [next(it) for _ in range(ns)]
        hs2r, d2r, b2r = next(it), next(it), next(it)
        wa, bar, out = next(it), next(it), next(it)

        acc = jnp.zeros((_RB, m), jnp.float32) + bar[...]
        for r, (qr, hsr, dr, br) in enumerate(
                ((q1r, hs1r, d1r, b1r), (q2r, hs2r, d2r, b2r))):
            for sl in range(ns):
                t = _elu(dr[...] * (qr[sl][0] + qr[sl][1] + hsr[sl])
                         + br[:, sl * 128:(sl + 1) * 128])
                acc += jnp.dot(t, wa[(r * ns + sl) * 128:(r * ns + sl + 1) * 128, :],
                               preferred_element_type=jnp.float32)
        out[...] = acc

    specs = ([_rows(_NUM_SC)((_RB, 128))] * ns
             + [_rows(ns)((_RB, 128)), _rows()((_RB, 1)), _full((1, ns * 128))])
    in_specs = (specs
                + [_rows(_NUM_SC)((_RB, 128))] * ns
                + [_rows(ns)((_RB, 128)), _rows()((_RB, 1)), _full((1, ns * 128))]
                + [_full((2 * ns * 128, m)), _full((1, m))])
    return _tc_call(
        body, nblk,
        jax.ShapeDtypeStruct((n, m), jnp.float32),
        in_specs,
        _rows()((_RB, m)),
    )(*q1, hs1, dv1, b1, *q2, hs2, dv2, b2, Wa, ba)


def kernel(node_feature, one_adj_list, two_adj_list,
                 W11, b11, W12, b12, W21, b21, W22, b22, W31, b31, W32, b32,
                 Wa1, ba1, Wa2, ba2, Wa3, ba3):
    n, f = node_feature.shape
    e = one_adj_list.shape[1]
    ew = e // _NW

    def _edges(adj):
        a = adj.astype(jnp.int32)
        return (a[0].reshape(_NW, ew // _B, _B), a[1].reshape(_NW, ew // _B, _B))

    src1, dst1 = _edges(one_adj_list)
    src2, dst2 = _edges(two_adj_list)

    def row2d(b):
        return b.reshape(1, -1)

    deg1 = _degrees(dst1, n, e)
    deg2 = _degrees(dst2, n, e)
    dv1, dv2, xs1, xs2 = _prep(deg1, deg2, node_feature, n)

    p1 = _segsum(src1, dst1, [xs1], n, e, f)[0]
    p2 = _segsum(src2, dst2, [xs2], n, e, f)[0]
    m1 = _layer1(p1, xs1, dv1, p2, xs2, dv2,
                 W11, row2d(b11), W12, row2d(b12), Wa1, row2d(ba1), n)

    hs1, hs2 = _branch_mm(m1, W21, W22, dv1, dv2, n)
    ns2 = hs1.shape[0]
    q1 = _segsum(src1, dst1, [hs1[i] for i in range(ns2)], n, e, 128)
    q2 = _segsum(src2, dst2, [hs2[i] for i in range(ns2)], n, e, 128)
    m2 = _combine_mm(q1, hs1, dv1, row2d(b21), q2, hs2, dv2, row2d(b22),
                     Wa2, row2d(ba2), n)

    h31, h32 = _branch_mm(m2, W31, W32, dv1, dv2, n)
    q31 = _segsum(src1, dst1, [h31[0]], n, e, 128)
    q32 = _segsum(src2, dst2, [h32[0]], n, e, 128)
    out = _combine_mm(q31, h31, dv1, row2d(b31), q32, h32, dv2, row2d(b32),
                      Wa3, row2d(ba3), n)
    return out


# speedup vs baseline: 14.5261x; 1.1963x over previous
import functools

import jax
import jax.numpy as jnp
from jax import lax
from jax.experimental import pallas as pl
from jax.experimental.pallas import tpu as pltpu
from jax.experimental.pallas import tpu_sc as plsc

_NUM_SC = 2
_NUM_TILES = 16
_NW = _NUM_SC * _NUM_TILES
_B = 100


def _elu(x):
    return jnp.where(x > 0.0, x, jnp.exp(x) - 1.0)


_NBUF = 2


@functools.lru_cache(maxsize=None)
def _make_segsum(n, e, d, num_slabs, gather):
    ew = e // _NW
    nb = ew // _B
    stripe = 640
    nacc = _NUM_TILES * stripe
    zr = 32
    nzc = stripe // zr
    ncols = d // 16

    mesh = plsc.VectorSubcoreMesh(
        core_axis_name="c", subcore_axis_name="s",
        num_cores=_NUM_SC, num_subcores=_NUM_TILES)

    def body(*refs):
        it = iter(refs)
        if gather:
            src_hbm = next(it)
        dst_hbm = next(it)
        h_hbm = [next(it) for _ in range(num_slabs)] if gather else []
        out_hbm = [next(it) for _ in range(num_slabs)]
        acc = next(it)
        if gather:
            src_v = next(it)
        dst_v = next(it)
        bufs = [next(it) for _ in range(_NBUF)]
        dbufs = [next(it) for _ in range(2)]
        zbuf = dbufs[0]
        gsems = [next(it) for _ in range(_NBUF)]
        ssems = [next(it) for _ in range(_NBUF)]
        zsem = next(it)
        dsems = [next(it) for _ in range(2)]
        osems = [next(it) for _ in range(2)]

        c = lax.axis_index("c")
        s = lax.axis_index("s")
        w = c * _NUM_TILES + s
        base_row = s * stripe

        zero16 = jnp.zeros((16,), jnp.float32)

        if not gather:
            one16 = jnp.ones((16,), jnp.float32)
            for i in range(_B):
                for j in range(ncols):
                    bufs[0][i, pl.ds(j * 16, 16)] = one16

        if gather:
            pltpu.sync_copy(src_hbm.at[w], src_v)
        pltpu.sync_copy(dst_hbm.at[w], dst_v)

        for slab in range(num_slabs):
            for i in range(zr):
                for j in range(ncols):
                    zbuf[i, pl.ds(j * 16, 16)] = zero16
            zds = [pltpu.async_copy(zbuf, acc.at[pl.ds(base_row + k * zr, zr)],
                                    zsem)
                   for k in range(nzc)]
            for zd in zds:
                zd.wait()
            plsc.subcore_barrier()

            if gather:
                h_slab = h_hbm[slab]
                gd = [None] * _NBUF
                sd = [None] * _NBUF

                def issue_scatter(jj):
                    bb = jj % _NBUF
                    gd[bb].wait()
                    sd[bb] = pltpu.async_copy(
                        bufs[bb], acc.at[dst_v.at[jj]], ssems[bb], add=True)

                for j in range(nb):
                    b = j % _NBUF
                    if sd[b] is not None:
                        sd[b].wait()
                        sd[b] = None
                    gd[b] = pltpu.async_copy(h_slab.at[src_v.at[j]],
                                             bufs[b], gsems[b])
                    if j >= _NBUF - 1:
                        issue_scatter(j - (_NBUF - 1))
                for jj in range(max(0, nb - _NBUF + 1), nb):
                    issue_scatter(jj)
                for b in range(_NBUF):
                    if sd[b] is not None:
                        sd[b].wait()
            else:
                sd = [None] * _NBUF
                for j in range(nb):
                    b = j % _NBUF
                    if sd[b] is not None:
                        sd[b].wait()
                    sd[b] = pltpu.async_copy(bufs[0], acc.at[dst_v.at[j]],
                                             ssems[b], add=True)
                for b in range(_NBUF):
                    if sd[b] is not None:
                        sd[b].wait()
            plsc.subcore_barrier()

            o_slab = out_hbm[slab]
            sdesc = [None, None]
            odesc = [None, None]
            for k in range(nzc):
                p = k % 2
                if odesc[p] is not None:
                    odesc[p].wait()
                sdesc[p] = pltpu.async_copy(
                    acc.at[pl.ds(base_row + k * zr, zr)], dbufs[p], dsems[p])
                sdesc[p].wait()
                odesc[p] = pltpu.async_copy(
                    dbufs[p], o_slab.at[c, pl.ds(base_row + k * zr, zr)],
                    osems[p])
            for od in odesc:
                if od is not None:
                    od.wait()

    out_type = [jax.ShapeDtypeStruct((_NUM_SC, nacc, d), jnp.float32)
                for _ in range(num_slabs)]
    scratch = [pltpu.VMEM_SHARED((nacc, d), jnp.float32)]
    if gather:
        scratch.append(pltpu.VMEM((nb, _B), jnp.int32))
    scratch += [pltpu.VMEM((nb, _B), jnp.int32)]
    scratch += [pltpu.VMEM((_B, d), jnp.float32) for _ in range(_NBUF)]
    scratch += [pltpu.VMEM((zr, d), jnp.float32) for _ in range(2)]
    scratch += [pltpu.SemaphoreType.DMA] * (2 * _NBUF + 5)
    return pl.kernel(body, out_type=out_type, mesh=mesh, scratch_types=scratch)


def _segsum(src, dst, h_slabs, n, e, d):
    fn = _make_segsum(n, e, d, len(h_slabs), True)
    return [o[:, :n, :] for o in fn(src, dst, *h_slabs)]


def _degrees(dst, n, e):
    fn = _make_segsum(n, e, 128, 1, False)
    return fn(dst)[0][:, :n, :]


_RB = 1000


def _full(shape):
    return pl.BlockSpec(shape, lambda i: (0,) * len(shape))


def _rows(*lead):
    def mk(shape):
        nl = len(lead)
        return pl.BlockSpec(tuple(lead) + shape,
                            lambda i: (0,) * nl + (i,) + (0,) * (len(shape) - 1))
    return mk


def _tc_call(body, nblk, out_shapes, in_specs, out_specs):
    return pl.pallas_call(
        body,
        grid=(nblk,),
        in_specs=in_specs,
        out_specs=out_specs,
        out_shape=out_shapes,
        compiler_params=pltpu.CompilerParams(
            dimension_semantics=("arbitrary",)),
    )


def _prep(deg1, deg2, x, n):
    def body(q1, q2, xr, dv1, dv2, xs1, xs2):
        d1 = lax.rsqrt(q1[0, :, :1] + q1[1, :, :1] + 1.0)
        d2 = lax.rsqrt(q2[0, :, :1] + q2[1, :, :1] + 1.0)
        dv1[...] = d1
        dv2[...] = d2
        xs1[...] = d1 * xr[...]
        xs2[...] = d2 * xr[...]

    nblk = n // _RB
    f = x.shape[1]
    return _tc_call(
        body, nblk,
        [jax.ShapeDtypeStruct((n, 1), jnp.float32),
         jax.ShapeDtypeStruct((n, 1), jnp.float32),
         jax.ShapeDtypeStruct((n, f), jnp.float32),
         jax.ShapeDtypeStruct((n, f), jnp.float32)],
        [_rows(_NUM_SC)((_RB, 128)), _rows(_NUM_SC)((_RB, 128)),
         _rows()((_RB, f))],
        [_rows()((_RB, 1)), _rows()((_RB, 1)),
         _rows()((_RB, f)), _rows()((_RB, f))],
    )(deg1, deg2, x)


def _layer1(p1, xs1, dv1, p2, xs2, dv2, W11, b11, W12, b12, Wa, ba, n):
    f = xs1.shape[1]
    k1 = W11.shape[1]
    m = Wa.shape[1]

    def body(p1r, xs1r, d1r, p2r, xs2r, d2r, w11, bb11, w12, bb12, wa, bba, out):
        a1 = d1r[...] * (p1r[0] + p1r[1] + xs1r[...])
        t1 = _elu(jnp.dot(a1, w11[...], preferred_element_type=jnp.float32)
                  + bb11[...])
        a2 = d2r[...] * (p2r[0] + p2r[1] + xs2r[...])
        t2 = _elu(jnp.dot(a2, w12[...], preferred_element_type=jnp.float32)
                  + bb12[...])
        out[...] = (jnp.dot(t1, wa[:k1, :], preferred_element_type=jnp.float32)
                    + jnp.dot(t2, wa[k1:, :], preferred_element_type=jnp.float32)
                    + bba[...])

    nblk = n // _RB
    return _tc_call(
        body, nblk,
        jax.ShapeDtypeStruct((n, m), jnp.float32),
        [_rows(_NUM_SC)((_RB, f)), _rows()((_RB, f)), _rows()((_RB, 1)),
         _rows(_NUM_SC)((_RB, f)), _rows()((_RB, f)), _rows()((_RB, 1)),
         _full((f, k1)), _full((1, k1)), _full((f, k1)), _full((1, k1)),
         _full((2 * k1, m)), _full((1, m))],
        _rows()((_RB, m)),
    )(p1, xs1, dv1, p2, xs2, dv2, W11, b11, W12, b12, Wa, ba)


def _branch_mm(m1, Wb1, Wb2, dv1, dv2, n):
    k = Wb1.shape[0]
    mout = Wb1.shape[1]
    ns = mout // 128
    nblk = n // _RB

    def body(mr, w1, w2, d1r, d2r, o1, o2):
        o1[0] = d1r[...] * jnp.dot(mr[...], w1[...],
                                   preferred_element_type=jnp.float32)
        o2[0] = d2r[...] * jnp.dot(mr[...], w2[...],
                                   preferred_element_type=jnp.float32)

    def colblk(shape):
        return pl.BlockSpec(shape, lambda i, j: (0, j))

    return pl.pallas_call(
        body,
        grid=(nblk, ns),
        in_specs=[pl.BlockSpec((_RB, k), lambda i, j: (i, 0)),
                  colblk((k, 128)), colblk((k, 128)),
                  pl.BlockSpec((_RB, 1), lambda i, j: (i, 0)),
                  pl.BlockSpec((_RB, 1), lambda i, j: (i, 0))],
        out_specs=[pl.BlockSpec((1, _RB, 128), lambda i, j: (j, i, 0)),
                   pl.BlockSpec((1, _RB, 128), lambda i, j: (j, i, 0))],
        out_shape=[jax.ShapeDtypeStruct((ns, n, 128), jnp.float32),
                   jax.ShapeDtypeStruct((ns, n, 128), jnp.float32)],
        compiler_params=pltpu.CompilerParams(
            dimension_semantics=("arbitrary", "arbitrary")),
    )(m1, Wb1, Wb2, dv1, dv2)


def _combine_mm(q1, hs1, dv1, b1, q2, hs2, dv2, b2, Wa, ba, n):
    ns = hs1.shape[0]
    m = Wa.shape[1]
    nblk = n // _RB

    def body(*refs):
        it = iter(refs)
        q1r = [next(it) for _ in range(ns)]
        hs1r, d1r, b1r = next(it), next(it), next(it)
        q2r = [next(it) for _ in range(ns)]
        hs2r, d2r, b2r = next(it), next(it), next(it)
        wa, bar, out = next(it), next(it), next(it)

        acc = jnp.zeros((_RB, m), jnp.float32) + bar[...]
        for r, (qr, hsr, dr, br) in enumerate(
                ((q1r, hs1r, d1r, b1r), (q2r, hs2r, d2r, b2r))):
            for sl in range(ns):
                t = _elu(dr[...] * (qr[sl][0] + qr[sl][1] + hsr[sl])
                         + br[:, sl * 128:(sl + 1) * 128])
                acc += jnp.dot(t, wa[(r * ns + sl) * 128:(r * ns + sl + 1) * 128, :],
                               preferred_element_type=jnp.float32)
        out[...] = acc

    specs = ([_rows(_NUM_SC)((_RB, 128))] * ns
             + [_rows(ns)((_RB, 128)), _rows()((_RB, 1)), _full((1, ns * 128))])
    in_specs = (specs
                + [_rows(_NUM_SC)((_RB, 128))] * ns
                + [_rows(ns)((_RB, 128)), _rows()((_RB, 1)), _full((1, ns * 128))]
                + [_full((2 * ns * 128, m)), _full((1, m))])
    return _tc_call(
        body, nblk,
        jax.ShapeDtypeStruct((n, m), jnp.float32),
        in_specs,
        _rows()((_RB, m)),
    )(*q1, hs1, dv1, b1, *q2, hs2, dv2, b2, Wa, ba)


def kernel(node_feature, one_adj_list, two_adj_list,
                 W11, b11, W12, b12, W21, b21, W22, b22, W31, b31, W32, b32,
                 Wa1, ba1, Wa2, ba2, Wa3, ba3):
    n, f = node_feature.shape
    e = one_adj_list.shape[1]
    ew = e // _NW

    def _edges(adj):
        a = adj.astype(jnp.int32)
        return (a[0].reshape(_NW, ew // _B, _B), a[1].reshape(_NW, ew // _B, _B))

    src1, dst1 = _edges(one_adj_list)
    src2, dst2 = _edges(two_adj_list)

    def row2d(b):
        return b.reshape(1, -1)

    deg1 = _degrees(dst1, n, e)
    deg2 = _degrees(dst2, n, e)
    dv1, dv2, xs1, xs2 = _prep(deg1, deg2, node_feature, n)

    p1 = _segsum(src1, dst1, [xs1], n, e, f)[0]
    p2 = _segsum(src2, dst2, [xs2], n, e, f)[0]
    m1 = _layer1(p1, xs1, dv1, p2, xs2, dv2,
                 W11, row2d(b11), W12, row2d(b12), Wa1, row2d(ba1), n)

    hs1, hs2 = _branch_mm(m1, W21, W22, dv1, dv2, n)
    ns2 = hs1.shape[0]
    q1 = _segsum(src1, dst1, [hs1[i] for i in range(ns2)], n, e, 128)
    q2 = _segsum(src2, dst2, [hs2[i] for i in range(ns2)], n, e, 128)
    m2 = _combine_mm(q1, hs1, dv1, row2d(b21), q2, hs2, dv2, row2d(b22),
                     Wa2, row2d(ba2), n)

    h31, h32 = _branch_mm(m2, W31, W32, dv1, dv2, n)
    q31 = _segsum(src1, dst1, [h31[0]], n, e, 128)
    q32 = _segsum(src2, dst2, [h32[0]], n, e, 128)
    out = _combine_mm(q31, h31, dv1, row2d(b31), q32, h32, dv2, row2d(b32),
                      Wa3, row2d(ba3), n)
    return out
